# Initial kernel scaffold; baseline (speedup 1.0000x reference)
import jax
import jax.numpy as jnp
from jax.experimental import pallas as pl


def kernel(x, edge_index, num_nodes):
    raise NotImplementedError("write your pallas kernel here")


# speedup vs baseline: 5.3481x; 5.3481x over previous
import math

import jax
import jax.numpy as jnp
from jax import lax
from jax.experimental import pallas as pl
from jax.experimental.pallas import tpu as pltpu
from jax.experimental.pallas import tpu_sc as plsc

NC = 2
NS = 16
L = 16


def _sc_partials(x, src_r, dst_r, h, ch, k):
    n, d = x.shape
    acc_rows = h + 128
    zt = acc_rows // NS
    mesh = plsc.VectorSubcoreMesh(core_axis_name="c", subcore_axis_name="s")
    zc = (k // 8) * 8
    zoffs = [(off, min(zc, zt - off)) for off in range(0, zt, zc)]

    def body(x_hbm, src_hbm, dst_hbm, out_hbm,
             src_v, dst_v, buf_a, buf_b, acc, sem_a, sem_b):
        cid = lax.axis_index("c")
        sid = lax.axis_index("s")
        pltpu.sync_copy(src_hbm.at[sid], src_v)
        pltpu.sync_copy(dst_hbm.at[sid], dst_v)

        zero = jnp.zeros((L,), jnp.float32)

        def zrow(j, carry):
            for t in range(0, d, L):
                buf_a[j, pl.ds(t, L)] = zero
            return carry

        lax.fori_loop(0, k, zrow, 0)
        zbase = pl.multiple_of(sid * zt, 8)
        for off, sz in zoffs:
            pltpu.sync_copy(buf_a.at[pl.ds(0, sz)], acc.at[pl.ds(zbase + off, sz)])

        base = cid * h

        def rmp(v):
            loc = v - base
            owned = loc.astype(jnp.uint32) < jnp.uint32(h)
            return jnp.where(owned, loc, h)

        tail = k % L != 0

        def remap(j, carry):
            if tail:
                vt = dst_v[j, pl.ds(k - L, L)]
            for t in range(0, k - L + 1, L):
                dst_v[j, pl.ds(t, L)] = rmp(dst_v[j, pl.ds(t, L)])
            if tail:
                dst_v[j, pl.ds(k - L, L)] = rmp(vt)
            return carry

        lax.fori_loop(0, ch, remap, 0)
        plsc.subcore_barrier()

        pltpu.async_copy(x_hbm.at[src_v.at[0]], buf_a, sem_a)
        pltpu.async_copy(x_hbm.at[src_v.at[1]], buf_b, sem_b)

        def step(i, carry):
            j0 = 2 * i
            for b, (buf, sem) in enumerate(((buf_a, sem_a), (buf_b, sem_b))):
                j = j0 + b
                pltpu.make_async_copy(x_hbm.at[src_v.at[j]], buf, sem).wait()
                pltpu.sync_copy(buf, acc.at[dst_v.at[j]], add=True)

                @pl.when(j + 2 < ch)
                def _():
                    pltpu.async_copy(x_hbm.at[src_v.at[j + 2]], buf, sem)
            return carry

        lax.fori_loop(0, ch // 2, step, 0)
        plsc.subcore_barrier()

        for off, sz in zoffs:
            pltpu.sync_copy(acc.at[pl.ds(zbase + off, sz)], buf_a.at[pl.ds(0, sz)])
            pltpu.sync_copy(buf_a.at[pl.ds(0, sz)],
                            out_hbm.at[cid, sid, pl.ds(off, sz)])

    call = pl.kernel(
        body,
        out_type=jax.ShapeDtypeStruct((NC, NS, zt, d), jnp.float32),
        mesh=mesh,
        scratch_types=[
            pltpu.VMEM((ch, k), jnp.int32),
            pltpu.VMEM((ch, k), jnp.int32),
            pltpu.VMEM((k, d), jnp.float32),
            pltpu.VMEM((k, d), jnp.float32),
            pltpu.VMEM_SHARED((acc_rows, d), jnp.float32),
            pltpu.SemaphoreType.DMA,
            pltpu.SemaphoreType.DMA,
        ],
    )
    return call(x, src_r, dst_r)


def _combine(p0, p1, nn, n, h):
    d = p0.shape[1]
    g = math.gcd(n, h)
    r = next((b for b in (2000, 1000, 400, 200, 80, 40, 16, 8) if g % b == 0), g)
    hb = h // r

    def body(nn_ref, a_ref, b_ref, o_ref):
        i = pl.program_id(0)
        rows = lax.broadcasted_iota(jnp.int32, (r, d), 0) + i * r
        s = jnp.where(i < hb, a_ref[...], b_ref[...])
        o_ref[...] = jnp.where(rows < nn_ref[0], s, 0.0)

    return pl.pallas_call(
        body,
        grid=(n // r,),
        in_specs=[
            pl.BlockSpec(memory_space=pltpu.SMEM),
            pl.BlockSpec((r, d), lambda i: (jnp.minimum(i, hb - 1), 0)),
            pl.BlockSpec((r, d), lambda i: (jnp.maximum(i - hb, 0), 0)),
        ],
        out_specs=pl.BlockSpec((r, d), lambda i: (i, 0)),
        out_shape=jax.ShapeDtypeStruct((n, d), jnp.float32),
    )(nn, p0, p1)


def kernel(x, edge_index, num_nodes):
    n, d = x.shape
    e = edge_index.shape[1]
    ei = edge_index.astype(jnp.int32)

    k = next(kk for kk in range(128, 15, -1)
             if e % (NS * kk) == 0 and (e // (NS * kk)) % 2 == 0)
    ch = e // (NS * k)
    er = ei.reshape(2, NS, ch, k)

    h = ((math.ceil(n / NC) + 127) // 128) * 128

    partials = _sc_partials(x, er[0], er[1], h, ch, k)
    p = partials.reshape(NC, h + 128, d)
    nn = jnp.reshape(num_nodes, (1,)).astype(jnp.int32)
    return _combine(p[0], p[1], nn, n, h)


# speedup vs baseline: 11.7468x; 2.1964x over previous
import math

import jax
import jax.numpy as jnp
from jax import lax
from jax.experimental import pallas as pl
from jax.experimental.pallas import tpu as pltpu
from jax.experimental.pallas import tpu_sc as plsc

NC = 2
NS = 16
NW = NC * NS
L = 16


def _sc_partials(x, src_r, dst_r, acc_rows, ch, k, csb):
    n, d = x.shape
    zt = acc_rows // NS
    mesh = plsc.VectorSubcoreMesh(core_axis_name="c", subcore_axis_name="s")
    zc = (k // 8) * 8
    zoffs = [(off, min(zc, zt - off)) for off in range(0, zt, zc)]

    def body(x_hbm, src_hbm, dst_hbm, out_hbm,
             src_v, dst_v, buf_a, buf_b, acc, sem_a, sem_b):
        cid = lax.axis_index("c")
        sid = lax.axis_index("s")
        wid = sid * NC + cid

        zero = jnp.zeros((L,), jnp.float32)

        def zrow(j, carry):
            for t in range(0, d, L):
                buf_a[j, pl.ds(t, L)] = zero
            return carry

        lax.fori_loop(0, k, zrow, 0)
        zbase = pl.multiple_of(sid * zt, 8)
        for off, sz in zoffs:
            pltpu.sync_copy(buf_a.at[pl.ds(0, sz)], acc.at[pl.ds(zbase + off, sz)])
        plsc.subcore_barrier()

        for sb in range(ch // csb):
            pltpu.sync_copy(src_hbm.at[wid, pl.ds(sb * csb, csb)], src_v)
            pltpu.sync_copy(dst_hbm.at[wid, pl.ds(sb * csb, csb)], dst_v)
            pltpu.async_copy(x_hbm.at[src_v.at[0]], buf_a, sem_a)
            pltpu.async_copy(x_hbm.at[src_v.at[1]], buf_b, sem_b)

            def step(i, carry):
                j0 = 2 * i
                for b, (buf, sem) in enumerate(((buf_a, sem_a), (buf_b, sem_b))):
                    j = j0 + b
                    pltpu.make_async_copy(x_hbm.at[src_v.at[j]], buf, sem).wait()
                    pltpu.sync_copy(buf, acc.at[dst_v.at[j]], add=True)

                    @pl.when(j + 2 < csb)
                    def _():
                        pltpu.async_copy(x_hbm.at[src_v.at[j + 2]], buf, sem)
                return carry

            lax.fori_loop(0, csb // 2, step, 0)
        plsc.subcore_barrier()

        for off, sz in zoffs:
            pltpu.sync_copy(acc.at[pl.ds(zbase + off, sz)], buf_a.at[pl.ds(0, sz)])
            pltpu.sync_copy(buf_a.at[pl.ds(0, sz)],
                            out_hbm.at[cid, sid, pl.ds(off, sz)])

    call = pl.kernel(
        body,
        out_type=jax.ShapeDtypeStruct((NC, NS, zt, d), jnp.float32),
        mesh=mesh,
        scratch_types=[
            pltpu.VMEM((csb, k), jnp.int32),
            pltpu.VMEM((csb, k), jnp.int32),
            pltpu.VMEM((k, d), jnp.float32),
            pltpu.VMEM((k, d), jnp.float32),
            pltpu.VMEM_SHARED((acc_rows, d), jnp.float32),
            pltpu.SemaphoreType.DMA,
            pltpu.SemaphoreType.DMA,
        ],
    )
    return call(x, src_r, dst_r)


def _combine(p0, p1, nn):
    n, d = p0.shape
    r = next((b for b in (2000, 1000, 400, 200, 80, 40, 16, 8) if n % b == 0), n)

    def body(nn_ref, a_ref, b_ref, o_ref):
        i = pl.program_id(0)
        rows = lax.broadcasted_iota(jnp.int32, (r, d), 0) + i * r
        s = a_ref[...] + b_ref[...]
        o_ref[...] = jnp.where(rows < nn_ref[0], s, 0.0)

    return pl.pallas_call(
        body,
        grid=(n // r,),
        in_specs=[
            pl.BlockSpec(memory_space=pltpu.SMEM),
            pl.BlockSpec((r, d), lambda i: (i, 0)),
            pl.BlockSpec((r, d), lambda i: (i, 0)),
        ],
        out_specs=pl.BlockSpec((r, d), lambda i: (i, 0)),
        out_shape=jax.ShapeDtypeStruct((n, d), jnp.float32),
    )(nn, p0, p1)


def kernel(x, edge_index, num_nodes):
    n, d = x.shape
    e = edge_index.shape[1]
    ei = edge_index.astype(jnp.int32)

    k = next(kk for kk in range(128, 15, -1)
             if e % (NW * kk) == 0 and (e // (NW * kk)) % 2 == 0)
    ch = e // (NW * k)
    er = ei.reshape(2, NW, ch, k)
    csb = next((c for c in range(min(48, ch), 7, -8)
                if ch % c == 0 and c % 2 == 0), ch)

    acc_rows = ((n + NS * 8 - 1) // (NS * 8)) * (NS * 8)

    partials = _sc_partials(x, er[0], er[1], acc_rows, ch, k, csb)
    p = partials.reshape(NC, acc_rows, d)[:, :n]
    nn = jnp.reshape(num_nodes, (1,)).astype(jnp.int32)
    return _combine(p[0], p[1], nn)
